# baseline (speedup 1.0000x reference)
import functools

import jax
import jax.numpy as jnp
from jax import lax
from jax.experimental import pallas as pl
from jax.experimental.pallas import tpu as pltpu
from jax.experimental.pallas import tpu_sc as plsc

B, C, H, W = 8, 4, 512, 512
NI = 16384
HW = H * W

NUM_CORES = 2
NUM_SUBCORES = 16
NW = NUM_CORES * NUM_SUBCORES

CH = 4096
CHUNKS_PER_SEG = NI // CH

_mesh = plsc.VectorSubcoreMesh(core_axis_name="c", subcore_axis_name="s")


@functools.partial(
    pl.kernel,
    mesh=_mesh,
    out_type=jax.ShapeDtypeStruct((NW, 16), jnp.float32),
    scratch_types=[
        pltpu.VMEM((4 * CH,), jnp.int32),
        pltpu.VMEM((CH,), jnp.int32),
        pltpu.VMEM((CH,), jnp.int32),
        pltpu.VMEM((CH,), jnp.float32),
        pltpu.VMEM((CH,), jnp.float32),
        pltpu.VMEM((16,), jnp.float32),
        pltpu.SemaphoreType.DMA,
    ],
)
def _loss_kernel(pred_hbm, iv0_hbm, iv1_hbm, out_hbm,
                 raw_v, idxb_v, idxd_v, valb_v, vald_v, acc_v, sem):
    wid = lax.axis_index("s") * NUM_CORES + lax.axis_index("c")
    lane = lax.iota(jnp.int32, 16)
    base = wid * HW
    c = wid % C

    def run_array(iv_hbm, g, acc):
        def chunk_body(ch, acc):
            off = (wid * CHUNKS_PER_SEG + ch) * (4 * CH)
            pltpu.sync_copy(iv_hbm.at[pl.ds(off, 4 * CH)], raw_v)

            def build(k, _):
                s = k * 16
                x0 = raw_v[pl.ds(s, 16)]
                y0 = raw_v[pl.ds(CH + s, 16)]
                x1 = raw_v[pl.ds(2 * CH + s, 16)]
                y1 = raw_v[pl.ds(3 * CH + s, 16)]
                idxb_v[pl.ds(s, 16)] = x0 * W + y0 + base
                idxd_v[pl.ds(s, 16)] = x1 * W + y1 + base
                return 0
            lax.fori_loop(0, CH // 16, build, 0)

            cb = pltpu.async_copy(pred_hbm.at[idxb_v], valb_v, sem)
            cd = pltpu.async_copy(pred_hbm.at[idxd_v], vald_v, sem)
            cb.wait()
            cd.wait()

            def accum(m, acc):
                s = pl.ds(m * 16, 16)
                d = valb_v[s] - vald_v[s]
                return acc + d * d
            acc = lax.fori_loop(0, CH // 16, accum, acc)

            vb0 = valb_v[pl.ds(0, 16)]
            vd0 = vald_v[pl.ds(0, 16)]
            d0 = vb0 - vd0
            d0 = d0 * d0
            g_eff = jnp.where(ch == 0, g, 0)
            acc = acc + jnp.where(lane < g_eff, 1.0 - 2.0 * d0, 0.0)
            return acc

        return lax.fori_loop(0, CHUNKS_PER_SEG, chunk_body, acc)

    g0 = jnp.where(c == 1, 2, jnp.where(c == 2, 3, 1))
    g1 = jnp.where(c == 1, 1, jnp.where(c == 2, 2, jnp.where(c == 3, 1, 0)))

    acc = jnp.zeros((16,), jnp.float32)
    acc = run_array(iv0_hbm, g0, acc)
    acc = run_array(iv1_hbm, g1, acc)

    acc_v[...] = acc
    pltpu.sync_copy(acc_v, out_hbm.at[wid])


def _rearrange(iv):
    return iv.reshape(-1, CH, 4).transpose(0, 2, 1).reshape(-1)


def kernel(prediction, intervals_comp_0, intervals_comp_1):
    pred_flat = prediction.reshape(-1)
    parts = _loss_kernel(pred_flat,
                         _rearrange(intervals_comp_0),
                         _rearrange(intervals_comp_1))
    return jnp.sum(parts)
